# baseline (speedup 1.0000x reference)
import functools

import jax
import jax.numpy as jnp
from jax import lax
from jax.experimental import pallas as pl
from jax.experimental.pallas import tpu as pltpu
from jax.experimental.pallas import tpu_sc as plsc

_NC = 2
_NS = 16
_NW = _NC * _NS
_CH = 128


def _sc_degree(dst2d, zeros16, n_acc):
    k = dst2d.shape[1]
    rows_sub = n_acc // _NS
    mesh = plsc.VectorSubcoreMesh(core_axis_name="c", subcore_axis_name="s")

    @functools.partial(
        pl.kernel,
        out_type=jax.ShapeDtypeStruct((_NC, n_acc, 16), jnp.float32),
        mesh=mesh,
        compiler_params=pltpu.CompilerParams(use_tc_tiling_on_sc=False),
        scratch_types=[
            pltpu.VMEM((k, _CH), jnp.int32),
            pltpu.VMEM((_CH, 16), jnp.float32),
            pltpu.VMEM_SHARED((n_acc, 16), jnp.float32),
        ],
    )
    def deg_kernel(dst_hbm, z_hbm, out_hbm, idx_v, ones_v, acc):
        cid = lax.axis_index("c")
        sid = lax.axis_index("s")
        wid = sid * _NC + cid

        @pl.loop(0, _CH)
        def _(i):
            ones_v[i, :] = jnp.full((16,), 1.0, jnp.float32)

        sub = pl.ds(sid * rows_sub, rows_sub)
        pltpu.sync_copy(z_hbm.at[sub], acc.at[sub])
        plsc.subcore_barrier()

        pltpu.sync_copy(dst_hbm.at[wid], idx_v)

        @pl.loop(0, k)
        def _(j):
            pltpu.sync_copy(ones_v, acc.at[idx_v.at[j]], add=True)

        plsc.subcore_barrier()
        pltpu.sync_copy(acc.at[sub], out_hbm.at[cid].at[sub])

    return deg_kernel(dst2d, zeros16)


_G = 8


def _sc_scatter(g, src4d, dst2d, zeros, n_acc):
    k = dst2d.shape[1]
    assert k % (2 * _G) == 0 and k // _G >= 2
    rows_sub = n_acc // _NS
    mesh = plsc.VectorSubcoreMesh(core_axis_name="c", subcore_axis_name="s")

    @functools.partial(
        pl.kernel,
        out_type=jax.ShapeDtypeStruct((_NC, n_acc, 128), jnp.float32),
        mesh=mesh,
        scratch_types=[
            pltpu.VMEM((2, _G, _CH), jnp.int32),
            pltpu.VMEM((k, _CH), jnp.int32),
            pltpu.VMEM((_CH, 128), jnp.float32),
            pltpu.VMEM((_CH, 128), jnp.float32),
            pltpu.VMEM_SHARED((n_acc, 128), jnp.float32),
            pltpu.SemaphoreType.DMA,
            pltpu.SemaphoreType.DMA,
            pltpu.SemaphoreType.DMA,
            pltpu.SemaphoreType.DMA,
        ],
    )
    def scat_kernel(g_hbm, src_hbm, dst_hbm, z_hbm, out_hbm,
                    isrc_v, idst_v, rows_a, rows_b, acc,
                    sem_ga, sem_gb, sem_i0, sem_i1):
        cid = lax.axis_index("c")
        sid = lax.axis_index("s")
        wid = sid * _NC + cid
        rows = (rows_a, rows_b)
        sem_g = (sem_ga, sem_gb)
        sem_i = (sem_i0, sem_i1)

        sub = pl.ds(sid * rows_sub, rows_sub)
        pltpu.sync_copy(z_hbm.at[sub], acc.at[sub])
        pltpu.sync_copy(dst_hbm.at[wid], idst_v)
        plsc.subcore_barrier()

        def do_group(base_c, s, has_next):
            for j in range(_G):
                pltpu.async_copy(g_hbm.at[isrc_v.at[s, j]],
                                 rows[j % 2], sem_g[j % 2]).wait()
                pltpu.sync_copy(rows[j % 2], acc.at[idst_v.at[base_c + j]],
                                add=True)
            if has_next:
                pltpu.make_async_copy(src_hbm.at[wid, 0],
                                      isrc_v.at[1 - s],
                                      sem_i[1 - s]).wait()

        ng = k // _G
        pltpu.async_copy(src_hbm.at[wid, 0], isrc_v.at[0], sem_i0)
        pltpu.async_copy(src_hbm.at[wid, 1], isrc_v.at[1], sem_i1)
        pltpu.make_async_copy(src_hbm.at[wid, 0], isrc_v.at[0],
                              sem_i0).wait()

        @pl.loop(0, (ng - 2) // 2)
        def _(gp):
            base = gp * (2 * _G)
            for p in range(2):
                do_group(base + p * _G, p, has_next=True)
                pltpu.async_copy(src_hbm.at[wid, 2 * gp + p + 2],
                                 isrc_v.at[p], sem_i[p])

        do_group(k - 2 * _G, 0, has_next=True)
        do_group(k - _G, 1, has_next=False)

        plsc.subcore_barrier()
        pltpu.sync_copy(acc.at[sub], out_hbm.at[cid].at[sub])

    return scat_kernel(g, src4d, dst2d, zeros)


def _tc_matmul(x, w):
    n = x.shape[0]
    blk = 1000

    def body(x_ref, w_ref, o_ref):
        o_ref[...] = jnp.dot(x_ref[...], w_ref[...],
                             preferred_element_type=jnp.float32)

    return pl.pallas_call(
        body,
        grid=(n // blk,),
        in_specs=[
            pl.BlockSpec((blk, x.shape[1]), lambda i: (i, 0)),
            pl.BlockSpec(w.shape, lambda i: (0, 0)),
        ],
        out_specs=pl.BlockSpec((blk, w.shape[1]), lambda i: (i, 0)),
        out_shape=jax.ShapeDtypeStruct((n, w.shape[1]), jnp.float32),
    )(x, w)


def _tc_scale(h1, degp):
    n, d = h1.shape
    blk = 1000

    def body(h_ref, deg_ref, g_ref, dv_ref):
        deg = deg_ref[0][:, 0:1] + deg_ref[1][:, 0:1] + 1.0
        dv = jnp.broadcast_to(lax.rsqrt(deg), (blk, d))
        g_ref[...] = h_ref[...] * dv
        dv_ref[...] = dv

    return pl.pallas_call(
        body,
        grid=(n // blk,),
        in_specs=[
            pl.BlockSpec((blk, d), lambda i: (i, 0)),
            pl.BlockSpec((2, blk, 16), lambda i: (0, i, 0)),
        ],
        out_specs=[
            pl.BlockSpec((blk, d), lambda i: (i, 0)),
            pl.BlockSpec((blk, d), lambda i: (i, 0)),
        ],
        out_shape=[
            jax.ShapeDtypeStruct((n, d), jnp.float32),
            jax.ShapeDtypeStruct((n, d), jnp.float32),
        ],
    )(h1, degp)


def _tc_mid(s1, g1, dvb, b1, w2):
    n, d = g1.shape
    blk = 1000

    def body(s_ref, g_ref, dv_ref, b_ref, w_ref, o_ref):
        h = dv_ref[...] * (s_ref[0] + s_ref[1] + g_ref[...]) + b_ref[...]
        h = jnp.maximum(h, 0.0)
        h2 = jnp.dot(h, w_ref[...], preferred_element_type=jnp.float32)
        o_ref[...] = h2 * dv_ref[...]

    return pl.pallas_call(
        body,
        grid=(n // blk,),
        in_specs=[
            pl.BlockSpec((2, blk, d), lambda i: (0, i, 0)),
            pl.BlockSpec((blk, d), lambda i: (i, 0)),
            pl.BlockSpec((blk, d), lambda i: (i, 0)),
            pl.BlockSpec((1, d), lambda i: (0, 0)),
            pl.BlockSpec(w2.shape, lambda i: (0, 0)),
        ],
        out_specs=pl.BlockSpec((blk, d), lambda i: (i, 0)),
        out_shape=jax.ShapeDtypeStruct((n, d), jnp.float32),
    )(s1, g1, dvb, b1, w2)


def _tc_final(s2, g2, dvb, b2):
    n, d = g2.shape
    blk = 1000

    def body(s_ref, g_ref, dv_ref, b_ref, o_ref):
        o = dv_ref[...] * (s_ref[0] + s_ref[1] + g_ref[...]) + b_ref[...]
        m = jnp.max(o, axis=1, keepdims=True)
        z = o - m
        lse = jnp.log(jnp.sum(jnp.exp(z), axis=1, keepdims=True))
        o_ref[...] = z - lse

    return pl.pallas_call(
        body,
        grid=(n // blk,),
        in_specs=[
            pl.BlockSpec((2, blk, d), lambda i: (0, i, 0)),
            pl.BlockSpec((blk, d), lambda i: (i, 0)),
            pl.BlockSpec((blk, d), lambda i: (i, 0)),
            pl.BlockSpec((1, d), lambda i: (0, 0)),
        ],
        out_specs=pl.BlockSpec((blk, d), lambda i: (i, 0)),
        out_shape=jax.ShapeDtypeStruct((n, d), jnp.float32),
    )(s2, g2, dvb, b2)


@jax.jit
def kernel(x, edge_index, eigenvectors, W1, b1, W2, b2):
    n, d_in = x.shape
    e = edge_index.shape[1]

    k = -(-e // (_NW * _CH))
    k = ((k + 2 * _G - 1) // (2 * _G)) * (2 * _G)
    e_pad = _NW * _CH * k
    src_p = jnp.concatenate(
        [edge_index[0], jnp.zeros((e_pad - e,), jnp.int32)])
    dst_p = jnp.concatenate(
        [edge_index[1], jnp.full((e_pad - e,), n, jnp.int32)])
    src4d = src_p.reshape(_NW, k // _G, _G, _CH)
    dst2d = dst_p.reshape(_NW, k, _CH)

    n_acc = ((n + 1 + _NS * 8 - 1) // (_NS * 8)) * (_NS * 8)
    z16 = jnp.zeros((n_acc, 16), jnp.float32)
    z128 = jnp.zeros((n_acc, 128), jnp.float32)

    degp = _sc_degree(dst2d, z16, n_acc)
    h1 = _tc_matmul(x, W1)
    g1, dvb = _tc_scale(h1, degp[:, :n, :])
    s1 = _sc_scatter(g1, src4d, dst2d, z128, n_acc)
    g2 = _tc_mid(s1[:, :n, :], g1, dvb, b1.reshape(1, -1), W2)
    s2 = _sc_scatter(g2, src4d, dst2d, z128, n_acc)
    return _tc_final(s2[:, :n, :], g2, dvb, b2.reshape(1, -1))


# speedup vs baseline: 1.1069x; 1.1069x over previous
import functools

import jax
import jax.numpy as jnp
from jax import lax
from jax.experimental import pallas as pl
from jax.experimental.pallas import tpu as pltpu
from jax.experimental.pallas import tpu_sc as plsc

_NC = 2
_NS = 16
_NW = _NC * _NS
_CH = 128


def _sc_degree(dst2d, zeros16, n_acc):
    k = dst2d.shape[1]
    rows_sub = n_acc // _NS
    mesh = plsc.VectorSubcoreMesh(core_axis_name="c", subcore_axis_name="s")

    @functools.partial(
        pl.kernel,
        out_type=jax.ShapeDtypeStruct((_NC, n_acc, 16), jnp.float32),
        mesh=mesh,
        compiler_params=pltpu.CompilerParams(use_tc_tiling_on_sc=False),
        scratch_types=[
            pltpu.VMEM((k, _CH), jnp.int32),
            pltpu.VMEM((_CH, 16), jnp.float32),
            pltpu.VMEM_SHARED((n_acc, 16), jnp.float32),
        ],
    )
    def deg_kernel(dst_hbm, z_hbm, out_hbm, idx_v, ones_v, acc):
        cid = lax.axis_index("c")
        sid = lax.axis_index("s")
        wid = sid * _NC + cid

        @pl.loop(0, _CH)
        def _(i):
            ones_v[i, :] = jnp.full((16,), 1.0, jnp.float32)

        sub = pl.ds(sid * rows_sub, rows_sub)
        pltpu.sync_copy(z_hbm.at[sub], acc.at[sub])
        plsc.subcore_barrier()

        pltpu.sync_copy(dst_hbm.at[wid], idx_v)

        @pl.loop(0, k)
        def _(j):
            pltpu.sync_copy(ones_v, acc.at[idx_v.at[j]], add=True)

        plsc.subcore_barrier()
        pltpu.sync_copy(acc.at[sub], out_hbm.at[cid].at[sub])

    return deg_kernel(dst2d, zeros16)


_NPH = 2


def _sc_scatter(g, src3d, dst3d, zeros, n_acc):
    kp = dst3d.shape[2]
    assert kp % 2 == 0 and kp >= 4
    rows_sub = n_acc // _NS
    mesh = plsc.VectorSubcoreMesh(core_axis_name="c", subcore_axis_name="s")

    @functools.partial(
        pl.kernel,
        out_type=jax.ShapeDtypeStruct((_NC, n_acc, 128), jnp.float32),
        mesh=mesh,
        scratch_types=[
            pltpu.VMEM((kp, _CH), jnp.int32),
            pltpu.VMEM((kp, _CH), jnp.int32),
            pltpu.VMEM((_CH, 128), jnp.float32),
            pltpu.VMEM((_CH, 128), jnp.float32),
            pltpu.VMEM_SHARED((n_acc, 128), jnp.float32),
            pltpu.SemaphoreType.DMA,
            pltpu.SemaphoreType.DMA,
        ],
    )
    def scat_kernel(g_hbm, src_hbm, dst_hbm, z_hbm, out_hbm,
                    isrc_v, idst_v, rows_a, rows_b, acc, sem_a, sem_b):
        cid = lax.axis_index("c")
        sid = lax.axis_index("s")
        wid = sid * _NC + cid

        sub = pl.ds(sid * rows_sub, rows_sub)
        pltpu.sync_copy(z_hbm.at[sub], acc.at[sub])
        plsc.subcore_barrier()

        def wait_a():
            pltpu.make_async_copy(g_hbm.at[isrc_v.at[0]], rows_a,
                                  sem_a).wait()

        def wait_b():
            pltpu.make_async_copy(g_hbm.at[isrc_v.at[0]], rows_b,
                                  sem_b).wait()

        for ph in range(_NPH):
            pltpu.sync_copy(src_hbm.at[wid, ph], isrc_v)
            pltpu.sync_copy(dst_hbm.at[wid, ph], idst_v)
            pltpu.async_copy(g_hbm.at[isrc_v.at[0]], rows_a, sem_a)

            @pl.loop(0, kp // 2 - 1)
            def _(t):
                c0 = 2 * t
                pltpu.async_copy(g_hbm.at[isrc_v.at[c0 + 1]], rows_b, sem_b)
                wait_a()
                pltpu.sync_copy(rows_a, acc.at[idst_v.at[c0]], add=True)
                pltpu.async_copy(g_hbm.at[isrc_v.at[c0 + 2]], rows_a, sem_a)
                wait_b()
                pltpu.sync_copy(rows_b, acc.at[idst_v.at[c0 + 1]], add=True)

            pltpu.async_copy(g_hbm.at[isrc_v.at[kp - 1]], rows_b, sem_b)
            wait_a()
            pltpu.sync_copy(rows_a, acc.at[idst_v.at[kp - 2]], add=True)
            wait_b()
            pltpu.sync_copy(rows_b, acc.at[idst_v.at[kp - 1]], add=True)

        plsc.subcore_barrier()
        pltpu.sync_copy(acc.at[sub], out_hbm.at[cid].at[sub])

    return scat_kernel(g, src3d, dst3d, zeros)


def _tc_matmul(x, w):
    n = x.shape[0]
    blk = 1000

    def body(x_ref, w_ref, o_ref):
        o_ref[...] = jnp.dot(x_ref[...], w_ref[...],
                             preferred_element_type=jnp.float32)

    return pl.pallas_call(
        body,
        grid=(n // blk,),
        in_specs=[
            pl.BlockSpec((blk, x.shape[1]), lambda i: (i, 0)),
            pl.BlockSpec(w.shape, lambda i: (0, 0)),
        ],
        out_specs=pl.BlockSpec((blk, w.shape[1]), lambda i: (i, 0)),
        out_shape=jax.ShapeDtypeStruct((n, w.shape[1]), jnp.float32),
    )(x, w)


def _tc_scale(h1, degp):
    n, d = h1.shape
    blk = 1000

    def body(h_ref, deg_ref, g_ref, dv_ref):
        deg = deg_ref[0][:, 0:1] + deg_ref[1][:, 0:1] + 1.0
        dv = jnp.broadcast_to(lax.rsqrt(deg), (blk, d))
        g_ref[...] = h_ref[...] * dv
        dv_ref[...] = dv

    return pl.pallas_call(
        body,
        grid=(n // blk,),
        in_specs=[
            pl.BlockSpec((blk, d), lambda i: (i, 0)),
            pl.BlockSpec((2, blk, 16), lambda i: (0, i, 0)),
        ],
        out_specs=[
            pl.BlockSpec((blk, d), lambda i: (i, 0)),
            pl.BlockSpec((blk, d), lambda i: (i, 0)),
        ],
        out_shape=[
            jax.ShapeDtypeStruct((n, d), jnp.float32),
            jax.ShapeDtypeStruct((n, d), jnp.float32),
        ],
    )(h1, degp)


def _tc_mid(s1, g1, dvb, b1, w2):
    n, d = g1.shape
    blk = 1000

    def body(s_ref, g_ref, dv_ref, b_ref, w_ref, o_ref):
        h = dv_ref[...] * (s_ref[0] + s_ref[1] + g_ref[...]) + b_ref[...]
        h = jnp.maximum(h, 0.0)
        h2 = jnp.dot(h, w_ref[...], preferred_element_type=jnp.float32)
        o_ref[...] = h2 * dv_ref[...]

    return pl.pallas_call(
        body,
        grid=(n // blk,),
        in_specs=[
            pl.BlockSpec((2, blk, d), lambda i: (0, i, 0)),
            pl.BlockSpec((blk, d), lambda i: (i, 0)),
            pl.BlockSpec((blk, d), lambda i: (i, 0)),
            pl.BlockSpec((1, d), lambda i: (0, 0)),
            pl.BlockSpec(w2.shape, lambda i: (0, 0)),
        ],
        out_specs=pl.BlockSpec((blk, d), lambda i: (i, 0)),
        out_shape=jax.ShapeDtypeStruct((n, d), jnp.float32),
    )(s1, g1, dvb, b1, w2)


def _tc_final(s2, g2, dvb, b2):
    n, d = g2.shape
    blk = 1000

    def body(s_ref, g_ref, dv_ref, b_ref, o_ref):
        o = dv_ref[...] * (s_ref[0] + s_ref[1] + g_ref[...]) + b_ref[...]
        m = jnp.max(o, axis=1, keepdims=True)
        z = o - m
        lse = jnp.log(jnp.sum(jnp.exp(z), axis=1, keepdims=True))
        o_ref[...] = z - lse

    return pl.pallas_call(
        body,
        grid=(n // blk,),
        in_specs=[
            pl.BlockSpec((2, blk, d), lambda i: (0, i, 0)),
            pl.BlockSpec((blk, d), lambda i: (i, 0)),
            pl.BlockSpec((blk, d), lambda i: (i, 0)),
            pl.BlockSpec((1, d), lambda i: (0, 0)),
        ],
        out_specs=pl.BlockSpec((blk, d), lambda i: (i, 0)),
        out_shape=jax.ShapeDtypeStruct((n, d), jnp.float32),
    )(s2, g2, dvb, b2)


@jax.jit
def kernel(x, edge_index, eigenvectors, W1, b1, W2, b2):
    n, d_in = x.shape
    e = edge_index.shape[1]

    k = -(-e // (_NW * _CH))
    k = ((k + 2 * _NPH - 1) // (2 * _NPH)) * (2 * _NPH)
    e_pad = _NW * _CH * k
    src_p = jnp.concatenate(
        [edge_index[0], jnp.zeros((e_pad - e,), jnp.int32)])
    dst_p = jnp.concatenate(
        [edge_index[1], jnp.full((e_pad - e,), n, jnp.int32)])
    src4d = src_p.reshape(_NW, _NPH, k // _NPH, _CH)
    dst2d = dst_p.reshape(_NW, _NPH, k // _NPH, _CH)

    n_acc = ((n + 1 + _NS * 8 - 1) // (_NS * 8)) * (_NS * 8)
    z16 = jnp.zeros((n_acc, 16), jnp.float32)
    z128 = jnp.zeros((n_acc, 128), jnp.float32)

    degp = _sc_degree(dst_p.reshape(_NW, k, _CH), z16, n_acc)
    h1 = _tc_matmul(x, W1)
    g1, dvb = _tc_scale(h1, degp[:, :n, :])
    s1 = _sc_scatter(g1, src4d, dst2d, z128, n_acc)
    g2 = _tc_mid(s1[:, :n, :], g1, dvb, b1.reshape(1, -1), W2)
    s2 = _sc_scatter(g2, src4d, dst2d, z128, n_acc)
    return _tc_final(s2[:, :n, :], g2, dvb, b2.reshape(1, -1))


# speedup vs baseline: 1.7602x; 1.5902x over previous
import functools

import jax
import jax.numpy as jnp
from jax import lax
from jax.experimental import pallas as pl
from jax.experimental.pallas import tpu as pltpu
from jax.experimental.pallas import tpu_sc as plsc

_NC = 2
_NS = 16
_NW = _NC * _NS
_CH = 128


def _sc_degree(dst2d, zeros16, n_acc):
    k = dst2d.shape[1]
    rows_sub = n_acc // _NS
    mesh = plsc.VectorSubcoreMesh(core_axis_name="c", subcore_axis_name="s")

    @functools.partial(
        pl.kernel,
        out_type=jax.ShapeDtypeStruct((_NC, n_acc, 16), jnp.float32),
        mesh=mesh,
        compiler_params=pltpu.CompilerParams(use_tc_tiling_on_sc=False),
        scratch_types=[
            pltpu.VMEM((k, _CH), jnp.int32),
            pltpu.VMEM((_CH, 16), jnp.float32),
            pltpu.VMEM_SHARED((n_acc, 16), jnp.float32),
        ],
    )
    def deg_kernel(dst_hbm, z_hbm, out_hbm, idx_v, ones_v, acc):
        cid = lax.axis_index("c")
        sid = lax.axis_index("s")
        wid = sid * _NC + cid

        @pl.loop(0, _CH)
        def _(i):
            ones_v[i, :] = jnp.full((16,), 1.0, jnp.float32)

        sub = pl.ds(sid * rows_sub, rows_sub)
        pltpu.sync_copy(z_hbm.at[sub], acc.at[sub])
        plsc.subcore_barrier()

        pltpu.sync_copy(dst_hbm.at[wid], idx_v)

        @pl.loop(0, k)
        def _(j):
            pltpu.sync_copy(ones_v, acc.at[idx_v.at[j]], add=True)

        plsc.subcore_barrier()
        pltpu.sync_copy(acc.at[sub], out_hbm.at[cid].at[sub])

    return deg_kernel(dst2d, zeros16)


def _sc_scatter(g, src2d, dst2d, n_acc):
    k = dst2d.shape[1]
    rows_sub = n_acc // _NS
    nz_full = rows_sub // _CH
    nz_tail = rows_sub - nz_full * _CH
    mesh = plsc.VectorSubcoreMesh(core_axis_name="c", subcore_axis_name="s")

    @functools.partial(
        pl.kernel,
        out_type=jax.ShapeDtypeStruct((_NC, n_acc, 128), jnp.float32),
        mesh=mesh,
        scratch_types=[
            pltpu.VMEM((k, _CH), jnp.int32),
            pltpu.VMEM((k, _CH), jnp.int32),
            pltpu.VMEM((_CH, 128), jnp.float32),
            pltpu.VMEM_SHARED((n_acc, 128), jnp.float32),
            pltpu.SemaphoreType.DMA,
            pltpu.SemaphoreType.DMA,
        ],
    )
    def scat_kernel(g_hbm, src_hbm, dst_hbm, out_hbm,
                    isrc_v, idst_v, rows_v, acc, sem, sem_z):
        cid = lax.axis_index("c")
        sid = lax.axis_index("s")
        wid = sid * _NC + cid

        @pl.loop(0, _CH)
        def _(i):
            for q in range(8):
                rows_v[i, pl.ds(q * 16, 16)] = jnp.zeros((16,), jnp.float32)

        base = sid * rows_sub
        for d in range(nz_full):
            pltpu.async_copy(rows_v, acc.at[pl.ds(base + d * _CH, _CH)],
                             sem_z)
        if nz_tail:
            pltpu.async_copy(
                rows_v.at[pl.ds(0, nz_tail)],
                acc.at[pl.ds(base + nz_full * _CH, nz_tail)], sem_z)
        pltpu.sync_copy(src_hbm.at[wid], isrc_v)
        pltpu.sync_copy(dst_hbm.at[wid], idst_v)
        for d in range(nz_full):
            pltpu.make_async_copy(rows_v,
                                  acc.at[pl.ds(base + d * _CH, _CH)],
                                  sem_z).wait()
        if nz_tail:
            pltpu.make_async_copy(
                rows_v.at[pl.ds(0, nz_tail)],
                acc.at[pl.ds(base + nz_full * _CH, nz_tail)], sem_z).wait()
        plsc.subcore_barrier()

        @pl.loop(0, k)
        def _(j):
            pltpu.async_copy(g_hbm.at[isrc_v.at[j]], rows_v, sem).wait()
            pltpu.sync_copy(rows_v, acc.at[idst_v.at[j]], add=True)

        plsc.subcore_barrier()
        sub = pl.ds(base, rows_sub)
        pltpu.sync_copy(acc.at[sub], out_hbm.at[cid].at[sub])

    return scat_kernel(g, src2d, dst2d)


def _tc_matmul(x, w):
    n = x.shape[0]
    blk = 1000

    def body(x_ref, w_ref, o_ref):
        o_ref[...] = jnp.dot(x_ref[...], w_ref[...],
                             preferred_element_type=jnp.float32)

    return pl.pallas_call(
        body,
        grid=(n // blk,),
        in_specs=[
            pl.BlockSpec((blk, x.shape[1]), lambda i: (i, 0)),
            pl.BlockSpec(w.shape, lambda i: (0, 0)),
        ],
        out_specs=pl.BlockSpec((blk, w.shape[1]), lambda i: (i, 0)),
        out_shape=jax.ShapeDtypeStruct((n, w.shape[1]), jnp.float32),
    )(x, w)


def _tc_scale(h1, degp):
    n, d = h1.shape
    blk = 1000

    def body(h_ref, deg_ref, g_ref, dv_ref):
        deg = deg_ref[0][:, 0:1] + deg_ref[1][:, 0:1] + 1.0
        dv = jnp.broadcast_to(lax.rsqrt(deg), (blk, d))
        g_ref[...] = h_ref[...] * dv
        dv_ref[...] = dv

    return pl.pallas_call(
        body,
        grid=(n // blk,),
        in_specs=[
            pl.BlockSpec((blk, d), lambda i: (i, 0)),
            pl.BlockSpec((2, blk, 16), lambda i: (0, i, 0)),
        ],
        out_specs=[
            pl.BlockSpec((blk, d), lambda i: (i, 0)),
            pl.BlockSpec((blk, d), lambda i: (i, 0)),
        ],
        out_shape=[
            jax.ShapeDtypeStruct((n, d), jnp.float32),
            jax.ShapeDtypeStruct((n, d), jnp.float32),
        ],
    )(h1, degp)


def _tc_mid(s1, g1, dvb, b1, w2):
    n, d = g1.shape
    blk = 1000

    def body(s_ref, g_ref, dv_ref, b_ref, w_ref, o_ref):
        h = dv_ref[...] * (s_ref[0] + s_ref[1] + g_ref[...]) + b_ref[...]
        h = jnp.maximum(h, 0.0)
        h2 = jnp.dot(h, w_ref[...], preferred_element_type=jnp.float32)
        o_ref[...] = h2 * dv_ref[...]

    return pl.pallas_call(
        body,
        grid=(n // blk,),
        in_specs=[
            pl.BlockSpec((2, blk, d), lambda i: (0, i, 0)),
            pl.BlockSpec((blk, d), lambda i: (i, 0)),
            pl.BlockSpec((blk, d), lambda i: (i, 0)),
            pl.BlockSpec((1, d), lambda i: (0, 0)),
            pl.BlockSpec(w2.shape, lambda i: (0, 0)),
        ],
        out_specs=pl.BlockSpec((blk, d), lambda i: (i, 0)),
        out_shape=jax.ShapeDtypeStruct((n, d), jnp.float32),
    )(s1, g1, dvb, b1, w2)


def _tc_final(s2, g2, dvb, b2):
    n, d = g2.shape
    blk = 1000

    def body(s_ref, g_ref, dv_ref, b_ref, o_ref):
        o = dv_ref[...] * (s_ref[0] + s_ref[1] + g_ref[...]) + b_ref[...]
        m = jnp.max(o, axis=1, keepdims=True)
        z = o - m
        lse = jnp.log(jnp.sum(jnp.exp(z), axis=1, keepdims=True))
        o_ref[...] = z - lse

    return pl.pallas_call(
        body,
        grid=(n // blk,),
        in_specs=[
            pl.BlockSpec((2, blk, d), lambda i: (0, i, 0)),
            pl.BlockSpec((blk, d), lambda i: (i, 0)),
            pl.BlockSpec((blk, d), lambda i: (i, 0)),
            pl.BlockSpec((1, d), lambda i: (0, 0)),
        ],
        out_specs=pl.BlockSpec((blk, d), lambda i: (i, 0)),
        out_shape=jax.ShapeDtypeStruct((n, d), jnp.float32),
    )(s2, g2, dvb, b2)


@jax.jit
def kernel(x, edge_index, eigenvectors, W1, b1, W2, b2):
    n, d_in = x.shape
    e = edge_index.shape[1]

    k = -(-e // (_NW * _CH))
    e_pad = _NW * _CH * k
    src_p = jnp.concatenate(
        [edge_index[0], jnp.zeros((e_pad - e,), jnp.int32)])
    dst_p = jnp.concatenate(
        [edge_index[1], jnp.full((e_pad - e,), n, jnp.int32)])
    src2d = src_p.reshape(_NW, k, _CH)
    dst2d = dst_p.reshape(_NW, k, _CH)

    n_acc = ((n + 1 + _NS * 8 - 1) // (_NS * 8)) * (_NS * 8)
    z16 = jnp.zeros((n_acc, 16), jnp.float32)

    degp = _sc_degree(dst2d, z16, n_acc)
    h1 = _tc_matmul(x, W1)
    g1, dvb = _tc_scale(h1, degp[:, :n, :])
    s1 = _sc_scatter(g1, src2d, dst2d, n_acc)
    g2 = _tc_mid(s1[:, :n, :], g1, dvb, b1.reshape(1, -1), W2)
    s2 = _sc_scatter(g2, src2d, dst2d, n_acc)
    return _tc_final(s2[:, :n, :], g2, dvb, b2.reshape(1, -1))
